# baseline (speedup 1.0000x reference)
import jax
import jax.numpy as jnp
from jax import lax
from jax.experimental import pallas as pl
from jax.experimental.pallas import tpu as pltpu
from jax.experimental.pallas import tpu_sc as plsc

B, C, H, W = 8, 192, 112, 112
HO, WO = 2 * H, 2 * W
M = B * C
PIX = H * W
OPIX = HO * WO
NC, NS, L = 2, 16, 16
NW = NC * NS
CH = 2
CPIX = PIX // CH
CIDX = CPIX // 2
COPIX = OPIX // CH
NCHUNK = M * CH
PER_WC = NCHUNK // NW
HL = H // CH
GPR = W // (2 * L)


def _unpool_body(vals_hbm, idx_hbm, out_hbm,
                 val0, val1, idx0, idx1, img0, img1,
                 sv0, sv1, si0, si1, so0, so1):
    two_iota = lax.iota(jnp.int32, L) * 2
    zerov = jnp.zeros((L,), jnp.float32)
    wid = lax.axis_index("s") * NC + lax.axis_index("c")
    t0 = wid * PER_WC

    bufs = ((val0, idx0, img0, sv0, si0, so0),
            (val1, idx1, img1, sv1, si1, so1))

    pltpu.async_copy(vals_hbm.at[t0], val0, sv0)
    pltpu.async_copy(idx_hbm.at[t0], idx0, si0)

    def pair_loop(jj, carry):
        for P, (val_v, idx_v, img, sv, si, so) in enumerate(bufs):
            t = t0 + jj * 2 + P
            pltpu.make_async_copy(vals_hbm.at[t], val_v, sv).wait()
            pltpu.make_async_copy(idx_hbm.at[t], idx_v, si).wait()
            nval, nidx, _, nsv, nsi, _ = bufs[1 - P]
            if P == 0:
                pltpu.async_copy(vals_hbm.at[t + 1], nval, nsv)
                pltpu.async_copy(idx_hbm.at[t + 1], nidx, nsi)
            else:
                @pl.when(jj < PER_WC // 2 - 1)
                def _prefetch():
                    pltpu.async_copy(vals_hbm.at[t + 1], nval, nsv)
                    pltpu.async_copy(idx_hbm.at[t + 1], nidx, nsi)

            @pl.when(jj >= 1)
            def _wait_prev():
                pltpu.make_async_copy(img, out_hbm.at[t - 2], so).wait()

            def rowpair(hp, c):
                rb = hp * (4 * WO)
                ib = hp * (2 * W)
                for g in range(7):
                    packed = idx_v[pl.ds(ib // 2 + g * L, L)]
                    ua = packed & jnp.uint32(0xFFFF)
                    ub = packed >> jnp.uint32(16)
                    for q, iv_u in ((2 * g, ua), (2 * g + 1, ub)):
                        rp, w0 = q // 7, (q % 7) * L
                        iv = iv_u.astype(jnp.int32)
                        vv = val_v[pl.ds(ib + q * L, L)]
                        basev = two_iota + (rb + 2 * rp * WO + 2 * w0)
                        d = (iv - COPIX * P) - basev
                        plsc.store_scatter(
                            img, [basev], jnp.where(d == 0, vv, zerov))
                        plsc.store_scatter(
                            img, [basev + 1], jnp.where(d == 1, vv, zerov))
                        plsc.store_scatter(
                            img, [basev + WO], jnp.where(d == WO, vv, zerov))
                        plsc.store_scatter(
                            img, [basev + (WO + 1)],
                            jnp.where(d == WO + 1, vv, zerov))
                return c

            lax.fori_loop(0, HL // 2, rowpair, 0)
            pltpu.async_copy(img, out_hbm.at[t], so)
        return carry

    lax.fori_loop(0, PER_WC // 2, pair_loop, 0)
    tend = t0 + PER_WC
    pltpu.make_async_copy(img0, out_hbm.at[tend - 2], so0).wait()
    pltpu.make_async_copy(img1, out_hbm.at[tend - 1], so1).wait()


@jax.jit
def kernel(f_maps, indices):
    vals = f_maps.reshape(NCHUNK, CPIX)
    idxr = indices.reshape(NCHUNK, CPIX // 32, 2, L).astype(jnp.uint32)
    idx16 = (idxr[:, :, 0, :] | (idxr[:, :, 1, :] << 16)).reshape(NCHUNK, CIDX)
    mesh = plsc.VectorSubcoreMesh(
        core_axis_name="c", subcore_axis_name="s",
        num_cores=NC, num_subcores=NS,
    )
    out = pl.kernel(
        _unpool_body,
        out_type=jax.ShapeDtypeStruct((NCHUNK, COPIX), jnp.float32),
        mesh=mesh,
        scratch_types=[
            pltpu.VMEM((CPIX,), jnp.float32),
            pltpu.VMEM((CPIX,), jnp.float32),
            pltpu.VMEM((CIDX,), jnp.uint32),
            pltpu.VMEM((CIDX,), jnp.uint32),
            pltpu.VMEM((COPIX,), jnp.float32),
            pltpu.VMEM((COPIX,), jnp.float32),
            pltpu.SemaphoreType.DMA,
            pltpu.SemaphoreType.DMA,
            pltpu.SemaphoreType.DMA,
            pltpu.SemaphoreType.DMA,
            pltpu.SemaphoreType.DMA,
            pltpu.SemaphoreType.DMA,
        ],
        compiler_params=pltpu.CompilerParams(needs_layout_passes=False),
    )(vals, idx16)
    return out.reshape(B, C, HO, WO)


# speedup vs baseline: 4.9134x; 4.9134x over previous
import jax
import jax.numpy as jnp
from jax import lax
from jax.experimental import pallas as pl
from jax.experimental.pallas import tpu as pltpu
from jax.experimental.pallas import tpu_sc as plsc

B, C, H, W = 8, 192, 112, 112
HO, WO = 2 * H, 2 * W
M = B * C
PIX = H * W
OPIX = HO * WO
NC, NS, L = 2, 16, 16
NW = NC * NS
CH = 2
CPIX = PIX // CH
COPIX = OPIX // CH
NCHUNK = M * CH
PER_WC = NCHUNK // NW
HL = H // CH
GPR = W // L


def _unpool_body(vals_hbm, idx_hbm, out_hbm,
                 val0, val1, idx0, idx1, img0, img1, img2,
                 sv0, sv1, si0, si1, so0, so1, so2):
    wid = lax.axis_index("s") * NC + lax.axis_index("c")
    two_iota = lax.iota(jnp.int32, L) * 2
    zerov = jnp.zeros((L,), jnp.float32)
    t0 = wid * PER_WC

    inbufs = ((val0, sv0, idx0, si0), (val1, sv1, idx1, si1))
    imgbufs = ((img0, so0), (img1, so1), (img2, so2))

    pltpu.async_copy(vals_hbm.at[t0], val0, sv0)
    pltpu.async_copy(idx_hbm.at[t0], idx0, si0)

    def six_loop(jj, carry):
        for u in range(6):
            j = jj * 6 + u
            P = u % 2
            Q = u % 3
            t = t0 + j
            val_v, sv, idx_v, si = inbufs[P]
            img, so = imgbufs[Q]
            pltpu.make_async_copy(vals_hbm.at[t], val_v, sv).wait()
            pltpu.make_async_copy(idx_hbm.at[t], idx_v, si).wait()
            nval, nsv, nidx, nsi = inbufs[1 - P]
            if u < 5:
                pltpu.async_copy(vals_hbm.at[t + 1], nval, nsv)
                pltpu.async_copy(idx_hbm.at[t + 1], nidx, nsi)
            else:
                @pl.when(jj < PER_WC // 6 - 1)
                def _prefetch():
                    pltpu.async_copy(vals_hbm.at[t + 1], nval, nsv)
                    pltpu.async_copy(idx_hbm.at[t + 1], nidx, nsi)

            @pl.when(jj * 6 + u >= 3)
            def _wait_prev():
                pltpu.make_async_copy(img, out_hbm.at[t - 3], so).wait()

            def row(hl, c):
                rb = hl * (2 * WO)
                ib = hl * W
                for g in range(GPR):
                    iv = idx_v[pl.ds(ib + g * L, L)]
                    vv = val_v[pl.ds(ib + g * L, L)]
                    basev = two_iota + (rb + 2 * L * g)
                    d = (iv - COPIX * P) - basev
                    plsc.store_scatter(
                        img, [basev], jnp.where(d == 0, vv, zerov))
                    plsc.store_scatter(
                        img, [basev + 1], jnp.where(d == 1, vv, zerov))
                    plsc.store_scatter(
                        img, [basev + WO], jnp.where(d == WO, vv, zerov))
                    plsc.store_scatter(
                        img, [basev + (WO + 1)],
                        jnp.where(d == WO + 1, vv, zerov))
                return c

            lax.fori_loop(0, HL, row, 0)
            pltpu.async_copy(img, out_hbm.at[t], so)
        return carry

    lax.fori_loop(0, PER_WC // 6, six_loop, 0)
    tend = t0 + PER_WC
    pltpu.make_async_copy(img0, out_hbm.at[tend - 3], so0).wait()
    pltpu.make_async_copy(img1, out_hbm.at[tend - 2], so1).wait()
    pltpu.make_async_copy(img2, out_hbm.at[tend - 1], so2).wait()


@jax.jit
def kernel(f_maps, indices):
    vals = f_maps.reshape(NCHUNK, CPIX)
    idx = indices.reshape(NCHUNK, CPIX).astype(jnp.int32)
    mesh = plsc.VectorSubcoreMesh(
        core_axis_name="c", subcore_axis_name="s",
        num_cores=NC, num_subcores=NS,
    )
    out = pl.kernel(
        _unpool_body,
        out_type=jax.ShapeDtypeStruct((NCHUNK, COPIX), jnp.float32),
        mesh=mesh,
        scratch_types=[
            pltpu.VMEM((CPIX,), jnp.float32),
            pltpu.VMEM((CPIX,), jnp.float32),
            pltpu.VMEM((CPIX,), jnp.int32),
            pltpu.VMEM((CPIX,), jnp.int32),
            pltpu.VMEM((COPIX,), jnp.float32),
            pltpu.VMEM((COPIX,), jnp.float32),
            pltpu.VMEM((COPIX,), jnp.float32),
            pltpu.SemaphoreType.DMA,
            pltpu.SemaphoreType.DMA,
            pltpu.SemaphoreType.DMA,
            pltpu.SemaphoreType.DMA,
            pltpu.SemaphoreType.DMA,
            pltpu.SemaphoreType.DMA,
            pltpu.SemaphoreType.DMA,
        ],
        compiler_params=pltpu.CompilerParams(needs_layout_passes=False),
    )(vals, idx)
    return out.reshape(B, C, HO, WO)


# speedup vs baseline: 4.9373x; 1.0049x over previous
import jax
import jax.numpy as jnp
from jax import lax
from jax.experimental import pallas as pl
from jax.experimental.pallas import tpu as pltpu
from jax.experimental.pallas import tpu_sc as plsc

B, C, H, W = 8, 192, 112, 112
HO, WO = 2 * H, 2 * W
M = B * C
PIX = H * W
OPIX = HO * WO
NC, NS, L = 2, 16, 16
NW = NC * NS
CH = 2
CPIX = PIX // CH
COPIX = OPIX // CH
NCHUNK = M * CH
PER_WC = NCHUNK // NW
HL = H // CH
GPR = W // L


def _unpool_body(vals_hbm, idx_hbm, out_hbm,
                 val0, val1, idx0, idx1, img0, img1,
                 sv0, sv1, si0, si1, so0, so1):
    wid = lax.axis_index("s") * NC + lax.axis_index("c")
    two_iota = lax.iota(jnp.int32, L) * 2
    zerov = jnp.zeros((L,), jnp.float32)
    t0 = wid * PER_WC

    bufs = ((val0, idx0, img0, sv0, si0, so0),
            (val1, idx1, img1, sv1, si1, so1))

    pltpu.async_copy(vals_hbm.at[t0], val0, sv0)
    pltpu.async_copy(idx_hbm.at[t0], idx0, si0)

    def pair_loop(jj, carry):
        for P, (val_v, idx_v, img, sv, si, so) in enumerate(bufs):
            t = t0 + jj * 2 + P
            pltpu.make_async_copy(vals_hbm.at[t], val_v, sv).wait()
            pltpu.make_async_copy(idx_hbm.at[t], idx_v, si).wait()
            nval, nidx, _, nsv, nsi, _ = bufs[1 - P]
            if P == 0:
                pltpu.async_copy(vals_hbm.at[t + 1], nval, nsv)
                pltpu.async_copy(idx_hbm.at[t + 1], nidx, nsi)
            else:
                @pl.when(jj < PER_WC // 2 - 1)
                def _prefetch():
                    pltpu.async_copy(vals_hbm.at[t + 1], nval, nsv)
                    pltpu.async_copy(idx_hbm.at[t + 1], nidx, nsi)

            @pl.when(jj >= 1)
            def _wait_prev():
                pltpu.make_async_copy(img, out_hbm.at[t - 2], so).wait()

            def row(hl, c):
                rb = hl * (2 * WO)
                ib = hl * W
                for g in range(GPR):
                    iv = idx_v[pl.ds(ib + g * L, L)]
                    vv = val_v[pl.ds(ib + g * L, L)]
                    basev = two_iota + (rb + 2 * L * g)
                    d = (iv - COPIX * P) - basev
                    plsc.store_scatter(
                        img, [basev], jnp.where(d == 0, vv, zerov))
                    plsc.store_scatter(
                        img, [basev + 1], jnp.where(d == 1, vv, zerov))
                    plsc.store_scatter(
                        img, [basev + WO], jnp.where(d == WO, vv, zerov))
                    plsc.store_scatter(
                        img, [basev + (WO + 1)],
                        jnp.where(d == WO + 1, vv, zerov))
                return c

            lax.fori_loop(0, HL, row, 0)
            pltpu.async_copy(img, out_hbm.at[t], so)
        return carry

    lax.fori_loop(0, PER_WC // 2, pair_loop, 0)
    tend = t0 + PER_WC
    pltpu.make_async_copy(img0, out_hbm.at[tend - 2], so0).wait()
    pltpu.make_async_copy(img1, out_hbm.at[tend - 1], so1).wait()


@jax.jit
def kernel(f_maps, indices):
    vals = f_maps.reshape(NCHUNK, CPIX)
    idx = indices.reshape(NCHUNK, CPIX).astype(jnp.int32)
    mesh = plsc.VectorSubcoreMesh(
        core_axis_name="c", subcore_axis_name="s",
        num_cores=NC, num_subcores=NS,
    )
    out = pl.kernel(
        _unpool_body,
        out_type=jax.ShapeDtypeStruct((NCHUNK, COPIX), jnp.float32),
        mesh=mesh,
        scratch_types=[
            pltpu.VMEM((CPIX,), jnp.float32),
            pltpu.VMEM((CPIX,), jnp.float32),
            pltpu.VMEM((CPIX,), jnp.int32),
            pltpu.VMEM((CPIX,), jnp.int32),
            pltpu.VMEM((COPIX,), jnp.float32),
            pltpu.VMEM((COPIX,), jnp.float32),
            pltpu.SemaphoreType.DMA,
            pltpu.SemaphoreType.DMA,
            pltpu.SemaphoreType.DMA,
            pltpu.SemaphoreType.DMA,
            pltpu.SemaphoreType.DMA,
            pltpu.SemaphoreType.DMA,
        ],
        compiler_params=pltpu.CompilerParams(needs_layout_passes=False),
    )(vals, idx)
    return out.reshape(B, C, HO, WO)
